# baseline (speedup 1.0000x reference)
import jax
import jax.numpy as jnp
from jax.experimental import pallas as pl
from jax.experimental.pallas import tpu as pltpu

B, N, D = 128, 256, 128
DE, DA = 4, 32
H1, H2 = 128, 64


def _encoder_body(er_ref, adj_ref, nodes_ref, m_ref, mask_ref, be_ref, w1a_ref,
                  w1b_ref, b1_ref, w2a_ref, w2b_ref, b2_ref, out_ref):
    er = er_ref[0]
    adj = adj_ref[0]
    h0 = nodes_ref[0]
    y = jnp.dot(er, m_ref[...], preferred_element_type=jnp.float32)
    y3 = y.reshape(N, 8, N)
    logits = jnp.sum(y3 * mask_ref[...][None, :, :], axis=1)
    gate = jax.nn.sigmoid(logits + be_ref[0, 0])
    a = adj * gate
    m1 = jnp.dot(a, h0, preferred_element_type=jnp.float32)
    h1 = jnp.dot(h0, w1a_ref[...], preferred_element_type=jnp.float32)
    h1 += jnp.dot(m1, w1b_ref[...], preferred_element_type=jnp.float32)
    h1 = jax.nn.relu(h1 + b1_ref[...])
    m2 = jnp.dot(a, h1, preferred_element_type=jnp.float32)
    h2 = jnp.dot(h1, w2a_ref[...], preferred_element_type=jnp.float32)
    h2 += jnp.dot(m2, w2b_ref[...], preferred_element_type=jnp.float32)
    h2 = jax.nn.relu(h2 + b2_ref[...])
    out_ref[0, :, :] = jnp.sum(h2, axis=0, keepdims=True) * (1.0 / N)


def _critic_body(g_ref, act_ref, wo_ref, bo_ref, wc1a_ref, wc1b_ref, bc1_ref,
                 wc2_ref, bc2_ref, wv_ref, bv_ref, out_ref):
    g = g_ref[...]
    emb = jnp.dot(g, wo_ref[...], preferred_element_type=jnp.float32) + bo_ref[...]
    x = jnp.dot(emb, wc1a_ref[...], preferred_element_type=jnp.float32)
    x += jnp.dot(act_ref[...], wc1b_ref[...], preferred_element_type=jnp.float32)
    x = jax.nn.relu(x + bc1_ref[...])
    x = jax.nn.relu(jnp.dot(x, wc2_ref[...], preferred_element_type=jnp.float32)
                    + bc2_ref[...])
    out_ref[...] = jnp.dot(x, wv_ref[...], preferred_element_type=jnp.float32) \
        + bv_ref[...]


@jax.jit
def kernel(nodes, edges, adjacency, actions, We, be, W1, b1, W2, b2, Wo, bo,
           Wc1, bc1, Wc2, bc2, Wv, bv):
    er = edges.reshape(B, N * 8, 128)
    rows = jnp.arange(128)
    cols = jnp.arange(N)
    m = jnp.where((cols[None, :] % 32) == (rows[:, None] // DE),
                  We[rows % DE, 0][:, None], 0.0).astype(jnp.float32)
    mask = (jnp.arange(8)[:, None] == (cols[None, :] // 32)).astype(jnp.float32)

    w1a, w1b = W1[:D], W1[D:]
    w2a, w2b = W2[:H1], W2[H1:]

    gmean = pl.pallas_call(
        _encoder_body,
        grid=(B,),
        in_specs=[
            pl.BlockSpec((1, N * 8, 128), lambda b: (b, 0, 0)),
            pl.BlockSpec((1, N, N), lambda b: (b, 0, 0)),
            pl.BlockSpec((1, N, D), lambda b: (b, 0, 0)),
            pl.BlockSpec((128, N), lambda b: (0, 0)),
            pl.BlockSpec((8, N), lambda b: (0, 0)),
            pl.BlockSpec((1, 1), lambda b: (0, 0)),
            pl.BlockSpec((D, H1), lambda b: (0, 0)),
            pl.BlockSpec((D, H1), lambda b: (0, 0)),
            pl.BlockSpec((1, H1), lambda b: (0, 0)),
            pl.BlockSpec((H1, H2), lambda b: (0, 0)),
            pl.BlockSpec((H1, H2), lambda b: (0, 0)),
            pl.BlockSpec((1, H2), lambda b: (0, 0)),
        ],
        out_specs=pl.BlockSpec((1, 1, H2), lambda b: (b, 0, 0)),
        out_shape=jax.ShapeDtypeStruct((B, 1, H2), jnp.float32),
        compiler_params=pltpu.CompilerParams(
            dimension_semantics=("parallel",),
        ),
    )(er, adjacency, nodes, m, mask, be.reshape(1, 1), w1a, w1b,
      b1.reshape(1, H1), w2a, w2b, b2.reshape(1, H2))

    gmean = gmean.reshape(B, H2)
    wc1a, wc1b = Wc1[:H2], Wc1[H2:]

    q = pl.pallas_call(
        _critic_body,
        out_shape=jax.ShapeDtypeStruct((B, 1), jnp.float32),
    )(gmean, actions, Wo, bo.reshape(1, H2), wc1a, wc1b, bc1.reshape(1, H1),
      Wc2, bc2.reshape(1, H2), Wv, bv.reshape(1, 1))

    return q.reshape(B)


# speedup vs baseline: 2.3949x; 2.3949x over previous
import jax
import jax.numpy as jnp
from jax.experimental import pallas as pl
from jax.experimental.pallas import tpu as pltpu

B, N, D = 128, 256, 128
DE, DA = 4, 32
H1, H2 = 128, 64


def _encoder_body(er_ref, adj_ref, nodes_ref, w8_ref, be_ref, w1a_ref,
                  w1b_ref, b1_ref, w2a_ref, w2b_ref, b2_ref, out_ref):
    adj = adj_ref[0]
    h0 = nodes_ref[0]
    er3 = er_ref[0].reshape(N, 8, 128)
    prod = er3 * w8_ref[...][None, :, :]
    left = jnp.sum(prod[:, :DE, :], axis=1)
    right = jnp.sum(prod[:, DE:, :], axis=1)
    logits = jnp.concatenate([left, right], axis=-1)
    gate = jax.nn.sigmoid(logits + be_ref[0, 0])
    a = adj * gate
    m1 = jnp.dot(a, h0, preferred_element_type=jnp.float32)
    h1 = jnp.dot(h0, w1a_ref[...], preferred_element_type=jnp.float32)
    h1 += jnp.dot(m1, w1b_ref[...], preferred_element_type=jnp.float32)
    h1 = jax.nn.relu(h1 + b1_ref[...])
    m2 = jnp.dot(a, h1, preferred_element_type=jnp.float32)
    h2 = jnp.dot(h1, w2a_ref[...], preferred_element_type=jnp.float32)
    h2 += jnp.dot(m2, w2b_ref[...], preferred_element_type=jnp.float32)
    h2 = jax.nn.relu(h2 + b2_ref[...])
    out_ref[0, :, :] = jnp.sum(h2, axis=0, keepdims=True) * (1.0 / N)


def _critic_body(g_ref, act_ref, wo_ref, bo_ref, wc1a_ref, wc1b_ref, bc1_ref,
                 wc2_ref, bc2_ref, wv_ref, bv_ref, out_ref):
    g = g_ref[...]
    emb = jnp.dot(g, wo_ref[...], preferred_element_type=jnp.float32) + bo_ref[...]
    x = jnp.dot(emb, wc1a_ref[...], preferred_element_type=jnp.float32)
    x += jnp.dot(act_ref[...], wc1b_ref[...], preferred_element_type=jnp.float32)
    x = jax.nn.relu(x + bc1_ref[...])
    x = jax.nn.relu(jnp.dot(x, wc2_ref[...], preferred_element_type=jnp.float32)
                    + bc2_ref[...])
    out_ref[...] = jnp.dot(x, wv_ref[...], preferred_element_type=jnp.float32) \
        + bv_ref[...]


@jax.jit
def kernel(nodes, edges, adjacency, actions, We, be, W1, b1, W2, b2, Wo, bo,
           Wc1, bc1, Wc2, bc2, Wv, bv):
    er = (edges.reshape(B, N, 2, 128, DE)
          .transpose(0, 1, 2, 4, 3)
          .reshape(B, N * 8, 128))
    w8 = jnp.broadcast_to(jnp.tile(We[:, 0], 2)[:, None], (8, 128))

    w1a, w1b = W1[:D], W1[D:]
    w2a, w2b = W2[:H1], W2[H1:]

    gmean = pl.pallas_call(
        _encoder_body,
        grid=(B,),
        in_specs=[
            pl.BlockSpec((1, N * 8, 128), lambda b: (b, 0, 0)),
            pl.BlockSpec((1, N, N), lambda b: (b, 0, 0)),
            pl.BlockSpec((1, N, D), lambda b: (b, 0, 0)),
            pl.BlockSpec((8, 128), lambda b: (0, 0)),
            pl.BlockSpec((1, 1), lambda b: (0, 0)),
            pl.BlockSpec((D, H1), lambda b: (0, 0)),
            pl.BlockSpec((D, H1), lambda b: (0, 0)),
            pl.BlockSpec((1, H1), lambda b: (0, 0)),
            pl.BlockSpec((H1, H2), lambda b: (0, 0)),
            pl.BlockSpec((H1, H2), lambda b: (0, 0)),
            pl.BlockSpec((1, H2), lambda b: (0, 0)),
        ],
        out_specs=pl.BlockSpec((1, 1, H2), lambda b: (b, 0, 0)),
        out_shape=jax.ShapeDtypeStruct((B, 1, H2), jnp.float32),
        compiler_params=pltpu.CompilerParams(
            dimension_semantics=("parallel",),
        ),
    )(er, adjacency, nodes, w8, be.reshape(1, 1), w1a, w1b,
      b1.reshape(1, H1), w2a, w2b, b2.reshape(1, H2))

    gmean = gmean.reshape(B, H2)
    wc1a, wc1b = Wc1[:H2], Wc1[H2:]

    q = pl.pallas_call(
        _critic_body,
        out_shape=jax.ShapeDtypeStruct((B, 1), jnp.float32),
    )(gmean, actions, Wo, bo.reshape(1, H2), wc1a, wc1b, bc1.reshape(1, H1),
      Wc2, bc2.reshape(1, H2), Wv, bv.reshape(1, 1))

    return q.reshape(B)


# speedup vs baseline: 5.5454x; 2.3155x over previous
import jax
import jax.numpy as jnp
from jax.experimental import pallas as pl
from jax.experimental.pallas import tpu as pltpu

B, N, D = 128, 256, 128
DE, DA = 4, 32
H1, H2 = 128, 64
GPB = 16


def _body(er_ref, adj_ref, nodes_ref, we_ref, be_ref, w1_ref, b1_ref,
          w2_ref, b2_ref, act_ref, wo_ref, bo_ref, wc1_ref, bc1_ref,
          wc2_ref, bc2_ref, wv_ref, bv_ref, out_ref, gacc_ref):
    bidx = pl.program_id(0)
    for g in range(GPB):
        h0 = nodes_ref[g]
        logits_l = logits_r = 0.0
        for k in range(DE):
            wk = we_ref[k:k + 1, :]
            logits_l += er_ref[g, pl.Slice(k, N, 8), :] * wk
            logits_r += er_ref[g, pl.Slice(DE + k, N, 8), :] * wk
        be0 = be_ref[0, 0]
        adj = adj_ref[g]
        a_l = adj[:, :128] * jax.nn.sigmoid(logits_l + be0)
        a_r = adj[:, 128:] * jax.nn.sigmoid(logits_r + be0)
        a = jnp.concatenate([a_l, a_r], axis=-1)
        m1 = jnp.dot(a, h0, preferred_element_type=jnp.float32)
        h1 = jnp.dot(jnp.concatenate([h0, m1], axis=-1), w1_ref[...],
                     preferred_element_type=jnp.float32)
        h1 = jax.nn.relu(h1 + b1_ref[...])
        m2 = jnp.dot(a, h1, preferred_element_type=jnp.float32)
        h2 = jnp.dot(jnp.concatenate([h1, m2], axis=-1), w2_ref[...],
                     preferred_element_type=jnp.float32)
        h2 = jax.nn.relu(h2 + b2_ref[...])
        gacc_ref[pl.ds(bidx * GPB + g, 1), :] = \
            jnp.sum(h2, axis=0, keepdims=True) * (1.0 / N)

    @pl.when(bidx == B // GPB - 1)
    def _critic():
        gmean = gacc_ref[...]
        emb = jnp.dot(gmean, wo_ref[...],
                      preferred_element_type=jnp.float32) + bo_ref[...]
        x = jnp.dot(jnp.concatenate([emb, act_ref[...]], axis=-1),
                    wc1_ref[...], preferred_element_type=jnp.float32)
        x = jax.nn.relu(x + bc1_ref[...])
        x = jax.nn.relu(jnp.dot(x, wc2_ref[...],
                                preferred_element_type=jnp.float32)
                        + bc2_ref[...])
        out_ref[...] = jnp.dot(x, wv_ref[...],
                               preferred_element_type=jnp.float32) \
            + bv_ref[...]


@jax.jit
def kernel(nodes, edges, adjacency, actions, We, be, W1, b1, W2, b2, Wo, bo,
           Wc1, bc1, Wc2, bc2, Wv, bv):
    er = (edges.reshape(B, N, 2, 128, DE)
          .transpose(0, 1, 2, 4, 3)
          .reshape(B, N * 8, 128))
    we4 = jnp.broadcast_to(We.reshape(DE, 1), (DE, 128))

    const = lambda b: (0, 0)
    q = pl.pallas_call(
        _body,
        grid=(B // GPB,),
        in_specs=[
            pl.BlockSpec((GPB, N * 8, 128), lambda b: (b, 0, 0)),
            pl.BlockSpec((GPB, N, N), lambda b: (b, 0, 0)),
            pl.BlockSpec((GPB, N, D), lambda b: (b, 0, 0)),
            pl.BlockSpec((DE, 128), const),
            pl.BlockSpec((1, 1), const),
            pl.BlockSpec((2 * D, H1), const),
            pl.BlockSpec((1, H1), const),
            pl.BlockSpec((2 * H1, H2), const),
            pl.BlockSpec((1, H2), const),
            pl.BlockSpec((B, DA), const),
            pl.BlockSpec((H2, H2), const),
            pl.BlockSpec((1, H2), const),
            pl.BlockSpec((H2 + DA, H1), const),
            pl.BlockSpec((1, H1), const),
            pl.BlockSpec((H1, H2), const),
            pl.BlockSpec((1, H2), const),
            pl.BlockSpec((H2, 1), const),
            pl.BlockSpec((1, 1), const),
        ],
        out_specs=pl.BlockSpec((B, 1), const),
        out_shape=jax.ShapeDtypeStruct((B, 1), jnp.float32),
        scratch_shapes=[pltpu.VMEM((B, H2), jnp.float32)],
        compiler_params=pltpu.CompilerParams(
            dimension_semantics=("arbitrary",),
        ),
    )(er, adjacency, nodes, we4, be.reshape(1, 1), W1, b1.reshape(1, H1),
      W2, b2.reshape(1, H2), actions, Wo, bo.reshape(1, H2), Wc1,
      bc1.reshape(1, H1), Wc2, bc2.reshape(1, H2), Wv, bv.reshape(1, 1))

    return q.reshape(B)
